# Initial kernel scaffold; baseline (speedup 1.0000x reference)
import jax
import jax.numpy as jnp
from jax.experimental import pallas as pl


def kernel(x, pseudo, L_idx, W_edge, b_edge, W1_0, b1_0, W2_0, b2_0, gamma_0, beta_0, sigma_0, mu_0, W1_1, b1_1, W2_1, b2_1, gamma_1, beta_1, sigma_1, mu_1, fc1_W, fc1_b, fc2_W, fc2_b):
    raise NotImplementedError("write your pallas kernel here")


# speedup vs baseline: 1.4633x; 1.4633x over previous
import functools

import jax
import jax.numpy as jnp
from jax import lax
from jax.experimental import pallas as pl
from jax.experimental.pallas import tpu as pltpu
from jax.experimental.pallas import tpu_sc as plsc

M = 2048
NN = 16
B = 2
J = 4
ENC = 5
PK = 128

NC = 2
NS = 16
NW = NC * NS
E = M * NN
EPW = E // NW
NCHUNK = 2
CH = EPW // NCHUNK


def _make_spmm(F):
    mesh = plsc.VectorSubcoreMesh(core_axis_name="c", subcore_axis_name="s")

    @functools.partial(
        pl.kernel, mesh=mesh,
        out_type=jax.ShapeDtypeStruct((NC, M, PK), jnp.float32),
        scratch_types=[
            pltpu.VMEM((EPW,), jnp.int32),
            pltpu.VMEM((CH,), jnp.int32),
            pltpu.VMEM((CH,), jnp.int32),
            pltpu.VMEM((CH,), jnp.int32),
            pltpu.VMEM((CH,), jnp.int32),
            pltpu.VMEM((EPW,), jnp.float32),
            pltpu.VMEM((CH, PK), jnp.float32),
            pltpu.VMEM_SHARED((M, PK), jnp.float32),
            pltpu.SemaphoreType.DMA,
        ],
    )
    def spmm(lidx_hbm, a_hbm, hp_hbm, zeros_hbm, out_hbm,
             idx_v, n_v0, n_v1, m_v0, m_v1, a_v, rows, acc, sem):
        n_vs = (n_v0, n_v1)
        m_vs = (m_v0, m_v1)
        c = lax.axis_index("c")
        s = lax.axis_index("s")
        wid = c * NS + s
        base = wid * EPW
        rpt = M // NS

        pltpu.sync_copy(lidx_hbm.at[pl.ds(base, EPW)], idx_v)
        pltpu.sync_copy(a_hbm.at[pl.ds(base, EPW)], a_v)

        for ch in range(NCHUNK):
            def decomp(k, _, ch=ch):
                vec = idx_v[pl.ds(ch * CH + k * 16, 16)]
                n_vs[ch][pl.ds(k * 16, 16)] = lax.shift_right_logical(vec, 11)
                m_vs[ch][pl.ds(k * 16, 16)] = lax.bitwise_and(vec, 2047)
                return 0
            lax.fori_loop(0, CH // 16, decomp, 0)

        pltpu.sync_copy(zeros_hbm, acc.at[pl.ds(s * rpt, rpt)])
        plsc.subcore_barrier()

        for ch in range(NCHUNK):
            pltpu.async_copy(hp_hbm.at[m_vs[ch]], rows, sem).wait()

            def scale(k, _):
                av = a_v[pl.ds(ch * CH + k * 16, 16)]
                for j in range(16):
                    avj = lax.gather(
                        av, jnp.full((16, 1), j, jnp.int32),
                        lax.GatherDimensionNumbers(offset_dims=(),
                                                   collapsed_slice_dims=(0,),
                                                   start_index_map=(0,)),
                        (1,), mode=lax.GatherScatterMode.PROMISE_IN_BOUNDS)
                    i = k * 16 + j
                    for b in range(B):
                        for fc in range(F // 16):
                            sl = pl.ds(b * 64 + fc * 16, 16)
                            rows[i, sl] = rows[i, sl] * avj
                return 0
            lax.fori_loop(0, CH // 16, scale, 0)

            pltpu.sync_copy(rows, acc.at[n_vs[ch]], add=True)

        plsc.subcore_barrier()
        pltpu.sync_copy(acc.at[pl.ds(s * rpt, rpt)],
                        out_hbm.at[c].at[pl.ds(s * rpt, rpt)])

    return spmm


_spmm16 = _make_spmm(16)
_spmm32 = _make_spmm(32)


def _attn(pseudo, W_edge, b_edge, sigma, mu):
    embed = pseudo @ W_edge + b_edge
    w = jnp.zeros((embed.shape[0],), dtype=embed.dtype)
    for j in range(J):
        u = embed - mu[j]
        w = w + jnp.exp(-0.5 * jnp.sum(u * sigma[j] * u, axis=1))
    w2 = w.reshape(M, NN)
    e = jnp.exp(w2 - jnp.max(w2, axis=1, keepdims=True))
    return (e / jnp.sum(e, axis=1, keepdims=True)).reshape(-1)


def kernel(x, pseudo, L_idx, W_edge, b_edge, W1_0, b1_0, W2_0, b2_0, gamma_0, beta_0, sigma_0, mu_0, W1_1, b1_1, W2_1, b2_1, gamma_1, beta_1, sigma_1, mu_1, fc1_W, fc1_b, fc2_W, fc2_b):
    eid = jnp.arange(E, dtype=jnp.int32)
    wbuf = jnp.zeros((M * M,), dtype=jnp.int32).at[L_idx].max(eid + 1)
    winner = wbuf[L_idx] == eid + 1
    zeros = jnp.zeros((M // NS, PK), jnp.float32)

    layers = [(16, _spmm16, W1_0, b1_0, W2_0, b2_0, gamma_0, beta_0, sigma_0, mu_0),
              (32, _spmm32, W1_1, b1_1, W2_1, b2_1, gamma_1, beta_1, sigma_1, mu_1)]
    h = x
    for (F, spmm, W1, b1, W2, b2, gamma, beta, sigma, mu) in layers:
        a = jnp.where(winner, _attn(pseudo, W_edge, b_edge, sigma, mu), 0.0)
        hp = jnp.zeros((M, PK), jnp.float32)
        hp = hp.at[:, 0:F].set(h[0]).at[:, 64:64 + F].set(h[1])
        partial = spmm(L_idx, a, hp, zeros)
        psum = partial[0] + partial[1]
        Lx = jnp.stack([psum[:, 0:F], psum[:, 64:64 + F]])
        z = Lx @ W1 + b1 + h @ W2 + b2
        mean = jnp.mean(z, axis=(0, 1))
        var = jnp.var(z, axis=(0, 1))
        z = (z - mean) / jnp.sqrt(var + 1e-5) * gamma + beta
        h = jax.nn.relu(z)
    h = h.reshape(B, -1)
    h = jax.nn.relu(h @ fc1_W + fc1_b)
    return h @ fc2_W + fc2_b
